# baseline (speedup 1.0000x reference)
import math

import jax
import jax.numpy as jnp
import numpy as np
from jax import lax
from jax.experimental import pallas as pl
from jax.experimental.pallas import tpu as pltpu
from jax.experimental.pallas import tpu_sc as plsc

NUM_PLAYERS = 5
NUM_MOVES = 50
EMB_DIM = NUM_PLAYERS * NUM_MOVES
GAMMA = 0.9
LN_GAMMA = math.log(GAMMA)
ACTIVE_T = 2048
LANES = 16
PAD_DIM = 256
UNROLL = 8

_ZERO_ACC = np.zeros((LANES * PAD_DIM,), dtype=np.float32)


def _sc_body(mv_hbm, pv_hbm, off_hbm, zero_hbm, out_hbm,
             mv_v, pv_v, off_v, acc, red, sem):
    lane = lax.iota(jnp.int32, LANES)

    cpz = pltpu.async_copy(zero_hbm, acc, sem)
    cp1 = pltpu.async_copy(mv_hbm.at[pl.ds(0, ACTIVE_T)], mv_v, sem)
    cp2 = pltpu.async_copy(pv_hbm.at[pl.ds(0, ACTIVE_T)], pv_v, sem)
    cp3 = pltpu.async_copy(off_hbm, off_v, sem)
    cpz.wait()
    cp1.wait()
    cp2.wait()
    cp3.wait()

    off = off_v[...]
    row_base = lane * PAD_DIM

    def _scatter_block(j, carry):
        jb = j * (UNROLL * LANES)
        for u in range(UNROLL):
            o = jb + u * LANES
            mv = mv_v[pl.ds(o, LANES)]
            pv = pv_v[pl.ds(o, LANES)]
            adj = jnp.where(pv >= off, pv - off, pv + (NUM_PLAYERS - off))
            ids = adj * NUM_MOVES + mv
            t = (o + lane).astype(jnp.float32)
            fac = jnp.exp(t * jnp.float32(LN_GAMMA))
            plsc.addupdate_scatter(acc, [row_base + ids], fac)
        return carry

    lax.fori_loop(0, ACTIVE_T // (UNROLL * LANES), _scatter_block, 0)

    n_chunks = PAD_DIM // LANES

    def _row_add(r, carry):
        rb = r * PAD_DIM
        return tuple(
            carry[c] + acc[pl.ds(rb + c * LANES, LANES)] for c in range(n_chunks)
        )

    init = tuple(acc[pl.ds(c * LANES, LANES)] for c in range(n_chunks))
    sums = lax.fori_loop(1, LANES, _row_add, init)
    for c in range(n_chunks):
        red[pl.ds(c * LANES, LANES)] = sums[c]
    pltpu.sync_copy(red.at[pl.ds(0, EMB_DIM)], out_hbm)


def kernel(move_ids, players, cur_player_offset):
    mv = move_ids.astype(jnp.int32)
    pv = players.astype(jnp.int32)
    off = jnp.full((LANES,), cur_player_offset, dtype=jnp.int32)
    mesh = plsc.VectorSubcoreMesh(
        core_axis_name="c", subcore_axis_name="s", num_cores=1, num_subcores=1
    )
    f = pl.kernel(
        _sc_body,
        out_type=jax.ShapeDtypeStruct((EMB_DIM,), jnp.float32),
        mesh=mesh,
        compiler_params=pltpu.CompilerParams(needs_layout_passes=False),
        scratch_types=[
            pltpu.VMEM((ACTIVE_T,), jnp.int32),
            pltpu.VMEM((ACTIVE_T,), jnp.int32),
            pltpu.VMEM((LANES,), jnp.int32),
            pltpu.VMEM((LANES * PAD_DIM,), jnp.float32),
            pltpu.VMEM((PAD_DIM,), jnp.float32),
            pltpu.SemaphoreType.DMA,
        ],
    )
    return f(mv, pv, off, _ZERO_ACC)
